# baseline (speedup 1.0000x reference)
import functools

import numpy as np
import jax
import jax.numpy as jnp
from jax import lax
from jax.experimental import pallas as pl
from jax.experimental.pallas import tpu as pltpu
from jax.experimental.pallas import tpu_sc as plsc

F32 = jnp.float32


def _proj(x4, w4, b4):
    n4, d4 = x4.shape
    h4 = w4.shape[1]
    blk = n4

    def body(x_ref, w_ref, b_ref, o_ref, ob_ref):
        r = jnp.maximum(
            jnp.dot(x_ref[...], w_ref[...], preferred_element_type=F32)
            + b_ref[...], 0.0)
        o_ref[...] = r
        ob_ref[...] = r.astype(jnp.bfloat16)

    return pl.pallas_call(
        body,
        grid=(n4 // blk,),
        in_specs=[
            pl.BlockSpec((blk, d4), lambda i: (i, 0)),
            pl.BlockSpec((d4, h4), lambda i: (0, 0)),
            pl.BlockSpec((1, h4), lambda i: (0, 0)),
        ],
        out_specs=[pl.BlockSpec((blk, h4), lambda i: (i, 0)),
                   pl.BlockSpec((blk, h4), lambda i: (i, 0))],
        out_shape=[jax.ShapeDtypeStruct((n4, h4), F32),
                   jax.ShapeDtypeStruct((n4, h4), jnp.bfloat16)],
    )(x4, w4, b4)


def _msg(hs4, ea4, wk_stack, ek_stack, b_big):
    e4 = hs4.shape[0]
    dk = wk_stack.shape[0]
    ke = ea4.shape[1]
    dt = hs4.dtype
    blk = 2000

    def body(hs_ref, ea_ref, wk_ref, ek_ref, bb_ref, o_ref):
        hs = hs_ref[...]
        ea = ea_ref[...]
        acc = jnp.dot(hs, bb_ref[...], preferred_element_type=F32)
        for kk in range(dk):
            p = jnp.dot(hs, wk_ref[kk], preferred_element_type=F32)
            r = jnp.dot(ea, ek_ref[kk], preferred_element_type=F32)
            acc += p * r
        o_ref[...] = acc

    return pl.pallas_call(
        body,
        grid=(e4 // blk,),
        in_specs=[
            pl.BlockSpec((blk, 128), lambda i: (i, 0)),
            pl.BlockSpec((blk, ke), lambda i: (i, 0)),
            pl.BlockSpec((dk, 128, 128), lambda i: (0, 0, 0)),
            pl.BlockSpec((dk, ke, 128), lambda i: (0, 0, 0)),
            pl.BlockSpec((128, 128), lambda i: (0, 0)),
        ],
        out_specs=pl.BlockSpec((blk, 128), lambda i: (i, 0)),
        out_shape=jax.ShapeDtypeStruct((e4, 128), F32),
    )(hs4, ea4, wk_stack, ek_stack, b_big)


def _gru(aggp4, hid4, w_ih4, w_hh4, b_ih4, b_hh4, gbias4):
    n4, h4 = hid4.shape
    blk = n4

    def body(a_ref, h_ref, wi_ref, wh_ref, bi_ref, bh_ref, gb_ref, o_ref,
             ob_ref):
        hid = h_ref[...]
        x = jnp.maximum(a_ref[0] + a_ref[1] + gb_ref[...], 0.0)
        gi = jnp.dot(x, wi_ref[...], preferred_element_type=F32) + bi_ref[...]
        gh = jnp.dot(hid, wh_ref[...], preferred_element_type=F32) + bh_ref[...]
        r = jax.nn.sigmoid(gi[:, :h4] + gh[:, :h4])
        z = jax.nn.sigmoid(gi[:, h4:2 * h4] + gh[:, h4:2 * h4])
        nn = jnp.tanh(gi[:, 2 * h4:] + r * gh[:, 2 * h4:])
        res = (1.0 - z) * nn + z * hid
        o_ref[...] = res
        ob_ref[...] = res.astype(jnp.bfloat16)

    return pl.pallas_call(
        body,
        grid=(n4 // blk,),
        in_specs=[
            pl.BlockSpec((2, blk, h4), lambda i: (0, i, 0)),
            pl.BlockSpec((blk, h4), lambda i: (i, 0)),
            pl.BlockSpec((h4, 3 * h4), lambda i: (0, 0)),
            pl.BlockSpec((h4, 3 * h4), lambda i: (0, 0)),
            pl.BlockSpec((1, 3 * h4), lambda i: (0, 0)),
            pl.BlockSpec((1, 3 * h4), lambda i: (0, 0)),
            pl.BlockSpec((1, h4), lambda i: (0, 0)),
        ],
        out_specs=[pl.BlockSpec((blk, h4), lambda i: (i, 0)),
                   pl.BlockSpec((blk, h4), lambda i: (i, 0))],
        out_shape=[jax.ShapeDtypeStruct((n4, h4), F32),
                   jax.ShapeDtypeStruct((n4, h4), jnp.bfloat16)],
    )(aggp4, hid4, w_ih4, w_hh4, b_ih4, b_hh4, gbias4)


def _pool(h, h0, ids3, w_sp_h, w_sp_h0, b_sp, m_r, m_p, a_prelu):
    n, hh = h.shape
    blk = 1000
    ngrid = n // blk
    g = m_r.shape[1]
    b = m_r.shape[0]
    d = w_sp_h.shape[1]

    def body(h_ref, h0_ref, id_ref, wh_ref, wh0_ref, bs_ref, mr_ref, mp_ref,
             a_ref, o_ref, mh_ref, mh0_ref):
        i = pl.program_id(0)

        @pl.when(i == 0)
        def _init():
            mh_ref[...] = jnp.zeros_like(mh_ref)
            mh0_ref[...] = jnp.zeros_like(mh0_ref)

        ids = id_ref[0]
        gi = lax.broadcasted_iota(jnp.int32, (g, blk), 0)
        oh = (gi == ids).astype(F32)
        mh_ref[...] += jnp.dot(oh, h_ref[...], preferred_element_type=F32)
        mh0_ref[...] += jnp.dot(oh, h0_ref[...], preferred_element_type=F32)

        @pl.when(i == ngrid - 1)
        def _fin():
            rx = (jnp.dot(mh_ref[...], wh_ref[...], preferred_element_type=F32)
                  + jnp.dot(mh0_ref[...], wh0_ref[...], preferred_element_type=F32)
                  + bs_ref[...])
            rx = jnp.where(rx > 0, rx, a_ref[0, 0] * rx)
            o_ref[...] = jnp.concatenate(
                [jnp.dot(mr_ref[...], rx, preferred_element_type=F32),
                 jnp.dot(mp_ref[...], rx, preferred_element_type=F32)], axis=1)

    return pl.pallas_call(
        body,
        grid=(ngrid,),
        in_specs=[
            pl.BlockSpec((blk, hh), lambda i: (i, 0)),
            pl.BlockSpec((blk, hh), lambda i: (i, 0)),
            pl.BlockSpec((1, 1, blk), lambda i: (i, 0, 0)),
            pl.BlockSpec((hh, d), lambda i: (0, 0)),
            pl.BlockSpec((hh, d), lambda i: (0, 0)),
            pl.BlockSpec((1, d), lambda i: (0, 0)),
            pl.BlockSpec((b, g), lambda i: (0, 0)),
            pl.BlockSpec((b, g), lambda i: (0, 0)),
            pl.BlockSpec((1, 1), lambda i: (0, 0)),
        ],
        out_specs=pl.BlockSpec((b, 2 * d), lambda i: (0, 0)),
        out_shape=jax.ShapeDtypeStruct((b, 2 * d), F32),
        scratch_shapes=[pltpu.VMEM((g, hh), F32), pltpu.VMEM((g, hh), F32)],
    )(h, h0, ids3, w_sp_h, w_sp_h0, b_sp, m_r, m_p, a_prelu)


_NW = 32
_NC = 2
_NS = 16
_CH = 125
_GRP = 20


def _sc_gather(table, src2):
    n, h = table.shape
    dt = table.dtype
    nchunk = src2.shape[0]
    t_per = nchunk // _NW
    assert t_per % _GRP == 0
    mesh = plsc.VectorSubcoreMesh(core_axis_name="c", subcore_axis_name="s")

    g2 = _GRP // 2
    ngrp = t_per // g2

    @functools.partial(
        pl.kernel,
        out_type=jax.ShapeDtypeStruct((nchunk, _CH, h), dt),
        mesh=mesh,
        compiler_params=pltpu.CompilerParams(use_tc_tiling_on_sc=False),
        scratch_types=[
            pltpu.VMEM((t_per, _CH), jnp.int32),
            pltpu.VMEM((2 * g2, _CH, h), dt),
            pltpu.SemaphoreType.DMA,
            pltpu.SemaphoreType.DMA,
            pltpu.SemaphoreType.DMA,
        ],
    )
    def gather(table_hbm, src_hbm, out_hbm, idx_v, rows_v, sem0, sem1, semw):
        c = lax.axis_index("c")
        s = lax.axis_index("s")
        wid = s * _NC + c
        start = wid * t_per
        pltpu.sync_copy(src_hbm.at[pl.ds(start, t_per)], idx_v)
        sems = (sem0, sem1)

        def fire(g):
            buf = (g % 2) * g2
            return [pltpu.async_copy(
                table_hbm.at[idx_v.at[g * g2 + j]], rows_v.at[buf + j],
                sems[g % 2]) for j in range(g2)]

        gat = fire(0)
        wout = None
        for g in range(ngrp):
            nxt = None
            if g + 1 < ngrp:
                if wout is not None:
                    wout.wait()
                    wout = None
                nxt = fire(g + 1)
            for dsc in gat:
                dsc.wait()
            if wout is not None:
                wout.wait()
            wout = pltpu.async_copy(
                rows_v.at[pl.ds((g % 2) * g2, g2)],
                out_hbm.at[pl.ds(start + g * g2, g2)], semw)
            gat = nxt
        wout.wait()

    return gather(table, src2)


def _sc_scatter(msg3, dst2, zeros_nh):
    nchunk = msg3.shape[0]
    h = msg3.shape[2]
    n = zeros_nh.shape[0]
    t_per = nchunk // _NW
    assert t_per % _GRP == 0
    rows_per_sub = n // _NS
    mesh = plsc.VectorSubcoreMesh(core_axis_name="c", subcore_axis_name="s")

    g2 = _GRP // 2
    ngrp = t_per // g2

    @functools.partial(
        pl.kernel,
        out_type=jax.ShapeDtypeStruct((_NC * n, h), F32),
        mesh=mesh,
        compiler_params=pltpu.CompilerParams(use_tc_tiling_on_sc=False),
        scratch_types=[
            pltpu.VMEM((t_per, _CH), jnp.int32),
            pltpu.VMEM((2 * g2, _CH, h), F32),
            pltpu.VMEM_SHARED((n, h), F32),
            pltpu.SemaphoreType.DMA,
            pltpu.SemaphoreType.DMA,
            pltpu.SemaphoreType.DMA,
            pltpu.SemaphoreType.DMA,
        ],
    )
    def scatter(msg_hbm, dst_hbm, zero_hbm, out_hbm, idx_v, msg_v, acc_sh,
                sema0, sema1, seml0, seml1):
        c = lax.axis_index("c")
        s = lax.axis_index("s")
        row0 = s * rows_per_sub
        zdsc = pltpu.async_copy(zero_hbm.at[pl.ds(row0, rows_per_sub)],
                                acc_sh.at[pl.ds(row0, rows_per_sub)], sema0)
        start = (c * _NS + s) * t_per
        pltpu.sync_copy(dst_hbm.at[pl.ds(start, t_per)], idx_v)
        semas = (sema0, sema1)
        semls = (seml0, seml1)

        def load(g):
            return pltpu.async_copy(
                msg_hbm.at[pl.ds(start + g * g2, g2)],
                msg_v.at[pl.ds((g % 2) * g2, g2)], semls[g % 2])

        ld = load(0)
        zdsc.wait()
        plsc.subcore_barrier()
        adds = None
        for g in range(ngrp):
            nxt = None
            if g + 1 < ngrp:
                if adds is not None:
                    for dsc in adds:
                        dsc.wait()
                    adds = None
                nxt = load(g + 1)
            ld.wait()
            if adds is not None:
                for dsc in adds:
                    dsc.wait()
            buf = (g % 2) * g2
            adds = [pltpu.async_copy(
                msg_v.at[buf + j], acc_sh.at[idx_v.at[g * g2 + j]],
                semas[g % 2], add=True) for j in range(g2)]
            ld = nxt
        for dsc in adds:
            dsc.wait()
        plsc.subcore_barrier()
        pltpu.sync_copy(acc_sh.at[pl.ds(row0, rows_per_sub)],
                        out_hbm.at[pl.ds(c * n + row0, rows_per_sub)])

    return scatter(msg3, dst2, zeros_nh)


def kernel(node_attr, edge_index, edge_attr, node_to_graph, select_reactant,
           num_reactant_batch, num_product_batch,
           W_proj, b_proj, W_bond, b_bond, gnn_bias,
           W_ih, W_hh, b_ih, b_hh, W_sp, b_sp, prelu_a):
    n, d_node = node_attr.shape
    e = edge_index.shape[1]
    d_edge = edge_attr.shape[1]
    h = W_proj.shape[1]
    k = d_edge + 1
    b = num_reactant_batch.shape[0]
    g = select_reactant.shape[0]
    d_hid = W_sp.shape[1]

    nchunk = e // _CH
    src2 = edge_index[0].reshape(nchunk, _CH).astype(jnp.int32)
    dst2 = edge_index[1].reshape(nchunk, _CH).astype(jnp.int32)

    eye4 = jnp.eye(4, dtype=F32)
    wb = W_bond.reshape(d_edge, h, h)
    wk_stack = jax.vmap(lambda m: jnp.kron(eye4, m))(wb).astype(jnp.bfloat16)
    ek_stack = jnp.asarray(np.stack([
        np.kron(np.eye(4, dtype=np.float32),
                np.eye(d_edge, dtype=np.float32)[:, [kk]]
                @ np.ones((1, h), np.float32))
        for kk in range(d_edge)]), dtype=jnp.bfloat16)
    bb = b_bond.reshape(h, h)
    b_big = jax.scipy.linalg.block_diag(bb, bb, bb, bb).astype(jnp.bfloat16)

    w_proj4 = jax.scipy.linalg.block_diag(*([W_proj] * 4))
    b_proj4 = jnp.tile(b_proj, (4,)).reshape(1, 4 * h)
    w_ih3 = W_ih.T.reshape(h, 3, h)
    w_hh3 = W_hh.T.reshape(h, 3, h)
    w_ih4 = jnp.concatenate(
        [jax.scipy.linalg.block_diag(*([w_ih3[:, gg, :]] * 4)) for gg in range(3)],
        axis=1)
    w_hh4 = jnp.concatenate(
        [jax.scipy.linalg.block_diag(*([w_hh3[:, gg, :]] * 4)) for gg in range(3)],
        axis=1)
    b_ih4 = jnp.tile(b_ih.reshape(3, 1, h), (1, 4, 1)).reshape(1, 12 * h)
    b_hh4 = jnp.tile(b_hh.reshape(3, 1, h), (1, 4, 1)).reshape(1, 12 * h)
    gbias4 = jnp.tile(gnn_bias, (4,)).reshape(1, 4 * h)

    h0_4, h0b = _proj(node_attr.reshape(n // 4, 4 * d_node), w_proj4, b_proj4)

    zeros_nh = jnp.zeros((n, h), F32)

    hid4, hidb = h0_4, h0b
    ea4 = edge_attr.reshape(e // 4, 4 * d_edge).astype(jnp.bfloat16)
    for step in range(3):
        hsrc3 = _sc_gather(hidb.reshape(n, h), src2)
        msg4 = _msg(hsrc3.reshape(e // 4, 4 * h), ea4, wk_stack, ek_stack,
                    b_big)
        aggp4 = _sc_scatter(msg4.reshape(nchunk, _CH, h), dst2, zeros_nh)
        hid4, hidb = _gru(aggp4.reshape(2, n // 4, 4 * h), hid4, w_ih4,
                          w_hh4, b_ih4, b_hh4, gbias4)

    r_idx = jnp.nonzero(select_reactant, size=b)[0]
    p_idx = jnp.nonzero(jnp.logical_not(select_reactant), size=b)[0]
    seg_r = jnp.repeat(jnp.arange(b), num_reactant_batch, total_repeat_length=b)
    seg_p = jnp.repeat(jnp.arange(b), num_product_batch, total_repeat_length=b)
    ar = jnp.arange(b)[None, :]
    ag = jnp.arange(g)[None, :]
    m_r = jnp.dot((seg_r[:, None] == ar).astype(F32).T,
                  (r_idx[:, None] == ag).astype(F32))
    m_p = jnp.dot((seg_p[:, None] == ar).astype(F32).T,
                  (p_idx[:, None] == ag).astype(F32))

    ids3 = node_to_graph.astype(jnp.int32).reshape(n // 1000, 1, 1000)
    out = _pool(hid4.reshape(n, h), h0_4.reshape(n, h), ids3,
                W_sp[:h], W_sp[h:], b_sp.reshape(1, d_hid),
                m_r, m_p, jnp.reshape(prelu_a, (1, 1)))
    return out


# speedup vs baseline: 1.6021x; 1.6021x over previous
import functools

import numpy as np
import jax
import jax.numpy as jnp
from jax import lax
from jax.experimental import pallas as pl
from jax.experimental.pallas import tpu as pltpu
from jax.experimental.pallas import tpu_sc as plsc

F32 = jnp.float32


def _proj(x4, w4, b4):
    n4, d4 = x4.shape
    h4 = w4.shape[1]
    blk = n4

    def body(x_ref, w_ref, b_ref, o_ref):
        o_ref[...] = jnp.maximum(
            jnp.dot(x_ref[...], w_ref[...], preferred_element_type=F32)
            + b_ref[...], 0.0)

    return pl.pallas_call(
        body,
        grid=(n4 // blk,),
        in_specs=[
            pl.BlockSpec((blk, d4), lambda i: (i, 0)),
            pl.BlockSpec((d4, h4), lambda i: (0, 0)),
            pl.BlockSpec((1, h4), lambda i: (0, 0)),
        ],
        out_specs=pl.BlockSpec((blk, h4), lambda i: (i, 0)),
        out_shape=jax.ShapeDtypeStruct((n4, h4), F32),
    )(x4, w4, b4)


def _msg(hs4, ea4, wk_stack, ek_stack, b_big):
    e4 = hs4.shape[0]
    dk = wk_stack.shape[0]
    ke = ea4.shape[1]
    blk = 2000

    def body(hs_ref, ea_ref, wk_ref, ek_ref, bb_ref, o_ref):
        hs = hs_ref[...]
        ea = ea_ref[...]
        acc = jnp.dot(hs, bb_ref[...], preferred_element_type=F32)
        for kk in range(dk):
            p = jnp.dot(hs, wk_ref[kk], preferred_element_type=F32)
            r = jnp.dot(ea, ek_ref[kk], preferred_element_type=F32)
            acc += p * r
        o_ref[...] = acc

    return pl.pallas_call(
        body,
        grid=(e4 // blk,),
        in_specs=[
            pl.BlockSpec((blk, 128), lambda i: (i, 0)),
            pl.BlockSpec((blk, ke), lambda i: (i, 0)),
            pl.BlockSpec((dk, 128, 128), lambda i: (0, 0, 0)),
            pl.BlockSpec((dk, ke, 128), lambda i: (0, 0, 0)),
            pl.BlockSpec((128, 128), lambda i: (0, 0)),
        ],
        out_specs=pl.BlockSpec((blk, 128), lambda i: (i, 0)),
        out_shape=jax.ShapeDtypeStruct((e4, 128), F32),
    )(hs4, ea4, wk_stack, ek_stack, b_big)


def _gru(aggp4, aggp4b, hid4, w_ih4, w_hh4, b_ih4, b_hh4, gbias4):
    n4, h4 = hid4.shape
    blk = n4

    def body(a_ref, b_ref2, h_ref, wi_ref, wh_ref, bi_ref, bh_ref, gb_ref,
             o_ref):
        hid = h_ref[...]
        x = jnp.maximum(a_ref[0] + a_ref[1] + b_ref2[0] + b_ref2[1]
                        + gb_ref[...], 0.0)
        gi = jnp.dot(x, wi_ref[...], preferred_element_type=F32) + bi_ref[...]
        gh = jnp.dot(hid, wh_ref[...], preferred_element_type=F32) + bh_ref[...]
        r = jax.nn.sigmoid(gi[:, :h4] + gh[:, :h4])
        z = jax.nn.sigmoid(gi[:, h4:2 * h4] + gh[:, h4:2 * h4])
        nn = jnp.tanh(gi[:, 2 * h4:] + r * gh[:, 2 * h4:])
        o_ref[...] = (1.0 - z) * nn + z * hid

    return pl.pallas_call(
        body,
        grid=(n4 // blk,),
        in_specs=[
            pl.BlockSpec((2, blk, h4), lambda i: (0, i, 0)),
            pl.BlockSpec((2, blk, h4), lambda i: (0, i, 0)),
            pl.BlockSpec((blk, h4), lambda i: (i, 0)),
            pl.BlockSpec((h4, 3 * h4), lambda i: (0, 0)),
            pl.BlockSpec((h4, 3 * h4), lambda i: (0, 0)),
            pl.BlockSpec((1, 3 * h4), lambda i: (0, 0)),
            pl.BlockSpec((1, 3 * h4), lambda i: (0, 0)),
            pl.BlockSpec((1, h4), lambda i: (0, 0)),
        ],
        out_specs=pl.BlockSpec((blk, h4), lambda i: (i, 0)),
        out_shape=jax.ShapeDtypeStruct((n4, h4), F32),
    )(aggp4, aggp4b, hid4, w_ih4, w_hh4, b_ih4, b_hh4, gbias4)


def _pool(h, h0, ids3, w_sp_h, w_sp_h0, b_sp, m_r, m_p, a_prelu):
    n, hh = h.shape
    blk = 1000
    ngrid = n // blk
    g = m_r.shape[1]
    b = m_r.shape[0]
    d = w_sp_h.shape[1]

    def body(h_ref, h0_ref, id_ref, wh_ref, wh0_ref, bs_ref, mr_ref, mp_ref,
             a_ref, o_ref, mh_ref, mh0_ref):
        i = pl.program_id(0)

        @pl.when(i == 0)
        def _init():
            mh_ref[...] = jnp.zeros_like(mh_ref)
            mh0_ref[...] = jnp.zeros_like(mh0_ref)

        ids = id_ref[0]
        gi = lax.broadcasted_iota(jnp.int32, (g, blk), 0)
        oh = (gi == ids).astype(F32)
        mh_ref[...] += jnp.dot(oh, h_ref[...], preferred_element_type=F32)
        mh0_ref[...] += jnp.dot(oh, h0_ref[...], preferred_element_type=F32)

        @pl.when(i == ngrid - 1)
        def _fin():
            rx = (jnp.dot(mh_ref[...], wh_ref[...], preferred_element_type=F32)
                  + jnp.dot(mh0_ref[...], wh0_ref[...], preferred_element_type=F32)
                  + bs_ref[...])
            rx = jnp.where(rx > 0, rx, a_ref[0, 0] * rx)
            o_ref[...] = jnp.concatenate(
                [jnp.dot(mr_ref[...], rx, preferred_element_type=F32),
                 jnp.dot(mp_ref[...], rx, preferred_element_type=F32)], axis=1)

    return pl.pallas_call(
        body,
        grid=(ngrid,),
        in_specs=[
            pl.BlockSpec((blk, hh), lambda i: (i, 0)),
            pl.BlockSpec((blk, hh), lambda i: (i, 0)),
            pl.BlockSpec((1, 1, blk), lambda i: (i, 0, 0)),
            pl.BlockSpec((hh, d), lambda i: (0, 0)),
            pl.BlockSpec((hh, d), lambda i: (0, 0)),
            pl.BlockSpec((1, d), lambda i: (0, 0)),
            pl.BlockSpec((b, g), lambda i: (0, 0)),
            pl.BlockSpec((b, g), lambda i: (0, 0)),
            pl.BlockSpec((1, 1), lambda i: (0, 0)),
        ],
        out_specs=pl.BlockSpec((b, 2 * d), lambda i: (0, 0)),
        out_shape=jax.ShapeDtypeStruct((b, 2 * d), F32),
        scratch_shapes=[pltpu.VMEM((g, hh), F32), pltpu.VMEM((g, hh), F32)],
    )(h, h0, ids3, w_sp_h, w_sp_h0, b_sp, m_r, m_p, a_prelu)


_NW = 32
_NC = 2
_NS = 16
_CH = 125
_GRP = 20


def _sc_gather(table, src2):
    n, h = table.shape
    nchunk = src2.shape[0]
    t_per = nchunk // _NW
    assert t_per % _GRP == 0
    mesh = plsc.VectorSubcoreMesh(core_axis_name="c", subcore_axis_name="s")

    g2 = _GRP // 2
    ngrp = t_per // g2

    @functools.partial(
        pl.kernel,
        out_type=jax.ShapeDtypeStruct((nchunk, _CH, h), F32),
        mesh=mesh,
        compiler_params=pltpu.CompilerParams(use_tc_tiling_on_sc=False),
        scratch_types=[
            pltpu.VMEM((t_per, _CH), jnp.int32),
            pltpu.VMEM((2 * g2, _CH, h), F32),
            pltpu.SemaphoreType.DMA,
            pltpu.SemaphoreType.DMA,
            pltpu.SemaphoreType.DMA,
        ],
    )
    def gather(table_hbm, src_hbm, out_hbm, idx_v, rows_v, sem0, sem1, semw):
        c = lax.axis_index("c")
        s = lax.axis_index("s")
        wid = s * _NC + c
        start = wid * t_per
        pltpu.sync_copy(src_hbm.at[pl.ds(start, t_per)], idx_v)
        sems = (sem0, sem1)

        def fire(g):
            buf = (g % 2) * g2
            return [pltpu.async_copy(
                table_hbm.at[idx_v.at[g * g2 + j]], rows_v.at[buf + j],
                sems[g % 2]) for j in range(g2)]

        gat = fire(0)
        wout = None
        for g in range(ngrp):
            nxt = None
            if g + 1 < ngrp:
                if wout is not None:
                    wout.wait()
                    wout = None
                nxt = fire(g + 1)
            for dsc in gat:
                dsc.wait()
            if wout is not None:
                wout.wait()
            wout = pltpu.async_copy(
                rows_v.at[pl.ds((g % 2) * g2, g2)],
                out_hbm.at[pl.ds(start + g * g2, g2)], semw)
            gat = nxt
        wout.wait()

    return gather(table, src2)


def _sc_scatter(msg3, dst2, zeros_nh):
    nchunk = msg3.shape[0]
    h = msg3.shape[2]
    n = zeros_nh.shape[0]
    t_per = nchunk // _NW
    assert t_per % _GRP == 0
    rows_per_sub = n // _NS
    mesh = plsc.VectorSubcoreMesh(core_axis_name="c", subcore_axis_name="s")

    g2 = _GRP // 2
    ngrp = t_per // g2

    @functools.partial(
        pl.kernel,
        out_type=jax.ShapeDtypeStruct((_NC * n, h), F32),
        mesh=mesh,
        compiler_params=pltpu.CompilerParams(use_tc_tiling_on_sc=False),
        scratch_types=[
            pltpu.VMEM((t_per, _CH), jnp.int32),
            pltpu.VMEM((2 * g2, _CH, h), F32),
            pltpu.VMEM_SHARED((n, h), F32),
            pltpu.SemaphoreType.DMA,
            pltpu.SemaphoreType.DMA,
            pltpu.SemaphoreType.DMA,
            pltpu.SemaphoreType.DMA,
        ],
    )
    def scatter(msg_hbm, dst_hbm, zero_hbm, out_hbm, idx_v, msg_v, acc_sh,
                sema0, sema1, seml0, seml1):
        c = lax.axis_index("c")
        s = lax.axis_index("s")
        row0 = s * rows_per_sub
        zdsc = pltpu.async_copy(zero_hbm.at[pl.ds(row0, rows_per_sub)],
                                acc_sh.at[pl.ds(row0, rows_per_sub)], sema0)
        start = (c * _NS + s) * t_per
        pltpu.sync_copy(dst_hbm.at[pl.ds(start, t_per)], idx_v)
        semas = (sema0, sema1)
        semls = (seml0, seml1)

        def load(g):
            return pltpu.async_copy(
                msg_hbm.at[pl.ds(start + g * g2, g2)],
                msg_v.at[pl.ds((g % 2) * g2, g2)], semls[g % 2])

        ld = load(0)
        zdsc.wait()
        plsc.subcore_barrier()
        adds = None
        for g in range(ngrp):
            nxt = None
            if g + 1 < ngrp:
                if adds is not None:
                    for dsc in adds:
                        dsc.wait()
                    adds = None
                nxt = load(g + 1)
            ld.wait()
            if adds is not None:
                for dsc in adds:
                    dsc.wait()
            buf = (g % 2) * g2
            adds = [pltpu.async_copy(
                msg_v.at[buf + j], acc_sh.at[idx_v.at[g * g2 + j]],
                semas[g % 2], add=True) for j in range(g2)]
            ld = nxt
        for dsc in adds:
            dsc.wait()
        plsc.subcore_barrier()
        pltpu.sync_copy(acc_sh.at[pl.ds(row0, rows_per_sub)],
                        out_hbm.at[pl.ds(c * n + row0, rows_per_sub)])

    return scatter(msg3, dst2, zeros_nh)


def kernel(node_attr, edge_index, edge_attr, node_to_graph, select_reactant,
           num_reactant_batch, num_product_batch,
           W_proj, b_proj, W_bond, b_bond, gnn_bias,
           W_ih, W_hh, b_ih, b_hh, W_sp, b_sp, prelu_a):
    n, d_node = node_attr.shape
    e = edge_index.shape[1]
    d_edge = edge_attr.shape[1]
    h = W_proj.shape[1]
    k = d_edge + 1
    b = num_reactant_batch.shape[0]
    g = select_reactant.shape[0]
    d_hid = W_sp.shape[1]

    nchunk = e // _CH
    src2 = edge_index[0].reshape(nchunk, _CH).astype(jnp.int32)
    dst2 = edge_index[1].reshape(nchunk, _CH).astype(jnp.int32)

    eye4 = jnp.eye(4, dtype=F32)
    wb = W_bond.reshape(d_edge, h, h)
    wk_stack = jax.vmap(lambda m: jnp.kron(eye4, m))(wb)
    ek_stack = jnp.asarray(np.stack([
        np.kron(np.eye(4, dtype=np.float32),
                np.eye(d_edge, dtype=np.float32)[:, [kk]]
                @ np.ones((1, h), np.float32))
        for kk in range(d_edge)]))
    bb = b_bond.reshape(h, h)
    b_big = jax.scipy.linalg.block_diag(bb, bb, bb, bb)

    w_proj4 = jax.scipy.linalg.block_diag(*([W_proj] * 4))
    b_proj4 = jnp.tile(b_proj, (4,)).reshape(1, 4 * h)
    w_ih3 = W_ih.T.reshape(h, 3, h)
    w_hh3 = W_hh.T.reshape(h, 3, h)
    w_ih4 = jnp.concatenate(
        [jax.scipy.linalg.block_diag(*([w_ih3[:, gg, :]] * 4)) for gg in range(3)],
        axis=1)
    w_hh4 = jnp.concatenate(
        [jax.scipy.linalg.block_diag(*([w_hh3[:, gg, :]] * 4)) for gg in range(3)],
        axis=1)
    b_ih4 = jnp.tile(b_ih.reshape(3, 1, h), (1, 4, 1)).reshape(1, 12 * h)
    b_hh4 = jnp.tile(b_hh.reshape(3, 1, h), (1, 4, 1)).reshape(1, 12 * h)
    gbias4 = jnp.tile(gnn_bias, (4,)).reshape(1, 4 * h)

    h0_4 = _proj(node_attr.reshape(n // 4, 4 * d_node), w_proj4, b_proj4)

    zeros_nh = jnp.zeros((n, h), F32)

    hid4 = h0_4
    ea4 = edge_attr.reshape(e // 4, 4 * d_edge)
    half = nchunk // 2
    e4h = e // 8
    src_a, src_b = src2[:half], src2[half:]
    dst_a, dst_b = dst2[:half], dst2[half:]
    ea_a, ea_b = ea4[:e4h], ea4[e4h:]
    for step in range(3):
        tbl = hid4.reshape(n, h)
        ga = _sc_gather(tbl, src_a)
        gb = _sc_gather(tbl, src_b)
        ma = _msg(ga.reshape(e4h, 4 * h), ea_a, wk_stack, ek_stack, b_big)
        agg_a = _sc_scatter(ma.reshape(half, _CH, h), dst_a, zeros_nh)
        mb = _msg(gb.reshape(e4h, 4 * h), ea_b, wk_stack, ek_stack, b_big)
        agg_b = _sc_scatter(mb.reshape(half, _CH, h), dst_b, zeros_nh)
        hid4 = _gru(agg_a.reshape(2, n // 4, 4 * h),
                    agg_b.reshape(2, n // 4, 4 * h), hid4, w_ih4, w_hh4,
                    b_ih4, b_hh4, gbias4)

    r_idx = jnp.nonzero(select_reactant, size=b)[0]
    p_idx = jnp.nonzero(jnp.logical_not(select_reactant), size=b)[0]
    seg_r = jnp.repeat(jnp.arange(b), num_reactant_batch, total_repeat_length=b)
    seg_p = jnp.repeat(jnp.arange(b), num_product_batch, total_repeat_length=b)
    ar = jnp.arange(b)[None, :]
    ag = jnp.arange(g)[None, :]
    m_r = jnp.dot((seg_r[:, None] == ar).astype(F32).T,
                  (r_idx[:, None] == ag).astype(F32))
    m_p = jnp.dot((seg_p[:, None] == ar).astype(F32).T,
                  (p_idx[:, None] == ag).astype(F32))

    ids3 = node_to_graph.astype(jnp.int32).reshape(n // 1000, 1, 1000)
    out = _pool(hid4.reshape(n, h), h0_4.reshape(n, h), ids3,
                W_sp[:h], W_sp[h:], b_sp.reshape(1, d_hid),
                m_r, m_p, jnp.reshape(prelu_a, (1, 1)))
    return out


# speedup vs baseline: 1.7841x; 1.1136x over previous
import functools

import numpy as np
import jax
import jax.numpy as jnp
from jax import lax
from jax.experimental import pallas as pl
from jax.experimental.pallas import tpu as pltpu
from jax.experimental.pallas import tpu_sc as plsc

F32 = jnp.float32


def _proj(x4, w4, b4):
    n4, d4 = x4.shape
    h4 = w4.shape[1]
    blk = n4

    def body(x_ref, w_ref, b_ref, o_ref):
        o_ref[...] = jnp.maximum(
            jnp.dot(x_ref[...], w_ref[...], preferred_element_type=F32)
            + b_ref[...], 0.0)

    return pl.pallas_call(
        body,
        grid=(n4 // blk,),
        in_specs=[
            pl.BlockSpec((blk, d4), lambda i: (i, 0)),
            pl.BlockSpec((d4, h4), lambda i: (0, 0)),
            pl.BlockSpec((1, h4), lambda i: (0, 0)),
        ],
        out_specs=pl.BlockSpec((blk, h4), lambda i: (i, 0)),
        out_shape=jax.ShapeDtypeStruct((n4, h4), F32),
    )(x4, w4, b4)


def _msg(hs4, ea4, wk_stack, ek_stack, b_big):
    e4 = hs4.shape[0]
    dk = wk_stack.shape[0]
    ke = ea4.shape[1]
    blk = 4000

    def body(hs_ref, ea_ref, wk_ref, ek_ref, bb_ref, o_ref):
        hs = hs_ref[...]
        ea = ea_ref[...]
        acc = jnp.dot(hs, bb_ref[...], preferred_element_type=F32)
        for kk in range(dk):
            p = jnp.dot(hs, wk_ref[kk], preferred_element_type=F32)
            r = jnp.dot(ea, ek_ref[kk], preferred_element_type=F32)
            acc += p * r
        o_ref[...] = acc

    return pl.pallas_call(
        body,
        grid=(e4 // blk,),
        in_specs=[
            pl.BlockSpec((blk, 128), lambda i: (i, 0)),
            pl.BlockSpec((blk, ke), lambda i: (i, 0)),
            pl.BlockSpec((dk, 128, 128), lambda i: (0, 0, 0)),
            pl.BlockSpec((dk, ke, 128), lambda i: (0, 0, 0)),
            pl.BlockSpec((128, 128), lambda i: (0, 0)),
        ],
        out_specs=pl.BlockSpec((blk, 128), lambda i: (i, 0)),
        out_shape=jax.ShapeDtypeStruct((e4, 128), F32),
    )(hs4, ea4, wk_stack, ek_stack, b_big)


def _gru(aggp4, hid4, w_ih4, w_hh4, b_ih4, b_hh4, gbias4):
    n4, h4 = hid4.shape
    blk = n4

    def body(a_ref, h_ref, wi_ref, wh_ref, bi_ref, bh_ref, gb_ref, o_ref):
        hid = h_ref[...]
        x = jnp.maximum(a_ref[0] + a_ref[1] + gb_ref[...], 0.0)
        gi = jnp.dot(x, wi_ref[...], preferred_element_type=F32) + bi_ref[...]
        gh = jnp.dot(hid, wh_ref[...], preferred_element_type=F32) + bh_ref[...]
        r = jax.nn.sigmoid(gi[:, :h4] + gh[:, :h4])
        z = jax.nn.sigmoid(gi[:, h4:2 * h4] + gh[:, h4:2 * h4])
        nn = jnp.tanh(gi[:, 2 * h4:] + r * gh[:, 2 * h4:])
        o_ref[...] = (1.0 - z) * nn + z * hid

    return pl.pallas_call(
        body,
        grid=(n4 // blk,),
        in_specs=[
            pl.BlockSpec((2, blk, h4), lambda i: (0, i, 0)),
            pl.BlockSpec((blk, h4), lambda i: (i, 0)),
            pl.BlockSpec((h4, 3 * h4), lambda i: (0, 0)),
            pl.BlockSpec((h4, 3 * h4), lambda i: (0, 0)),
            pl.BlockSpec((1, 3 * h4), lambda i: (0, 0)),
            pl.BlockSpec((1, 3 * h4), lambda i: (0, 0)),
            pl.BlockSpec((1, h4), lambda i: (0, 0)),
        ],
        out_specs=pl.BlockSpec((blk, h4), lambda i: (i, 0)),
        out_shape=jax.ShapeDtypeStruct((n4, h4), F32),
    )(aggp4, hid4, w_ih4, w_hh4, b_ih4, b_hh4, gbias4)


def _pool(h, h0, ids3, w_sp_h, w_sp_h0, b_sp, m_r, m_p, a_prelu):
    n, hh = h.shape
    blk = 1000
    ngrid = n // blk
    g = m_r.shape[1]
    b = m_r.shape[0]
    d = w_sp_h.shape[1]

    def body(h_ref, h0_ref, id_ref, wh_ref, wh0_ref, bs_ref, mr_ref, mp_ref,
             a_ref, o_ref, mh_ref, mh0_ref):
        i = pl.program_id(0)

        @pl.when(i == 0)
        def _init():
            mh_ref[...] = jnp.zeros_like(mh_ref)
            mh0_ref[...] = jnp.zeros_like(mh0_ref)

        ids = id_ref[0]
        gi = lax.broadcasted_iota(jnp.int32, (g, blk), 0)
        oh = (gi == ids).astype(F32)
        mh_ref[...] += jnp.dot(oh, h_ref[...], preferred_element_type=F32)
        mh0_ref[...] += jnp.dot(oh, h0_ref[...], preferred_element_type=F32)

        @pl.when(i == ngrid - 1)
        def _fin():
            rx = (jnp.dot(mh_ref[...], wh_ref[...], preferred_element_type=F32)
                  + jnp.dot(mh0_ref[...], wh0_ref[...], preferred_element_type=F32)
                  + bs_ref[...])
            rx = jnp.where(rx > 0, rx, a_ref[0, 0] * rx)
            o_ref[...] = jnp.concatenate(
                [jnp.dot(mr_ref[...], rx, preferred_element_type=F32),
                 jnp.dot(mp_ref[...], rx, preferred_element_type=F32)], axis=1)

    return pl.pallas_call(
        body,
        grid=(ngrid,),
        in_specs=[
            pl.BlockSpec((blk, hh), lambda i: (i, 0)),
            pl.BlockSpec((blk, hh), lambda i: (i, 0)),
            pl.BlockSpec((1, 1, blk), lambda i: (i, 0, 0)),
            pl.BlockSpec((hh, d), lambda i: (0, 0)),
            pl.BlockSpec((hh, d), lambda i: (0, 0)),
            pl.BlockSpec((1, d), lambda i: (0, 0)),
            pl.BlockSpec((b, g), lambda i: (0, 0)),
            pl.BlockSpec((b, g), lambda i: (0, 0)),
            pl.BlockSpec((1, 1), lambda i: (0, 0)),
        ],
        out_specs=pl.BlockSpec((b, 2 * d), lambda i: (0, 0)),
        out_shape=jax.ShapeDtypeStruct((b, 2 * d), F32),
        scratch_shapes=[pltpu.VMEM((g, hh), F32), pltpu.VMEM((g, hh), F32)],
    )(h, h0, ids3, w_sp_h, w_sp_h0, b_sp, m_r, m_p, a_prelu)


_NW = 32
_NC = 2
_NS = 16
_CH = 125
_GRP = 20


def _sc_gather(table, src2):
    n, h = table.shape
    nchunk = src2.shape[0]
    t_per = nchunk // _NW
    assert t_per % _GRP == 0
    mesh = plsc.VectorSubcoreMesh(core_axis_name="c", subcore_axis_name="s")

    g2 = _GRP // 2
    ngrp = t_per // g2

    @functools.partial(
        pl.kernel,
        out_type=jax.ShapeDtypeStruct((nchunk, _CH, h), F32),
        mesh=mesh,
        compiler_params=pltpu.CompilerParams(use_tc_tiling_on_sc=False),
        scratch_types=[
            pltpu.VMEM((t_per, _CH), jnp.int32),
            pltpu.VMEM((2 * g2, _CH, h), F32),
            pltpu.SemaphoreType.DMA,
            pltpu.SemaphoreType.DMA,
            pltpu.SemaphoreType.DMA,
        ],
    )
    def gather(table_hbm, src_hbm, out_hbm, idx_v, rows_v, sem0, sem1, semw):
        c = lax.axis_index("c")
        s = lax.axis_index("s")
        wid = s * _NC + c
        start = wid * t_per
        pltpu.sync_copy(src_hbm.at[pl.ds(start, t_per)], idx_v)
        sems = (sem0, sem1)

        def fire(g):
            buf = (g % 2) * g2
            return [pltpu.async_copy(
                table_hbm.at[idx_v.at[g * g2 + j]], rows_v.at[buf + j],
                sems[g % 2]) for j in range(g2)]

        gat = fire(0)
        wout = None
        for g in range(ngrp):
            nxt = None
            if g + 1 < ngrp:
                if wout is not None:
                    wout.wait()
                    wout = None
                nxt = fire(g + 1)
            for dsc in gat:
                dsc.wait()
            if wout is not None:
                wout.wait()
            wout = pltpu.async_copy(
                rows_v.at[pl.ds((g % 2) * g2, g2)],
                out_hbm.at[pl.ds(start + g * g2, g2)], semw)
            gat = nxt
        wout.wait()

    return gather(table, src2)


def _sc_scatter(msg3, dst2, zeros_nh):
    nchunk = msg3.shape[0]
    h = msg3.shape[2]
    n = zeros_nh.shape[0]
    t_per = nchunk // _NW
    assert t_per % _GRP == 0
    rows_per_sub = n // _NS
    mesh = plsc.VectorSubcoreMesh(core_axis_name="c", subcore_axis_name="s")

    g2 = _GRP // 2
    ngrp = t_per // g2

    @functools.partial(
        pl.kernel,
        out_type=jax.ShapeDtypeStruct((_NC * n, h), F32),
        mesh=mesh,
        compiler_params=pltpu.CompilerParams(use_tc_tiling_on_sc=False),
        scratch_types=[
            pltpu.VMEM((t_per, _CH), jnp.int32),
            pltpu.VMEM((2 * g2, _CH, h), F32),
            pltpu.VMEM_SHARED((n, h), F32),
            pltpu.SemaphoreType.DMA,
            pltpu.SemaphoreType.DMA,
            pltpu.SemaphoreType.DMA,
            pltpu.SemaphoreType.DMA,
        ],
    )
    def scatter(msg_hbm, dst_hbm, zero_hbm, out_hbm, idx_v, msg_v, acc_sh,
                sema0, sema1, seml0, seml1):
        c = lax.axis_index("c")
        s = lax.axis_index("s")
        row0 = s * rows_per_sub
        zdsc = pltpu.async_copy(zero_hbm.at[pl.ds(row0, rows_per_sub)],
                                acc_sh.at[pl.ds(row0, rows_per_sub)], sema0)
        start = (c * _NS + s) * t_per
        pltpu.sync_copy(dst_hbm.at[pl.ds(start, t_per)], idx_v)
        semas = (sema0, sema1)
        semls = (seml0, seml1)

        def load(g):
            return pltpu.async_copy(
                msg_hbm.at[pl.ds(start + g * g2, g2)],
                msg_v.at[pl.ds((g % 2) * g2, g2)], semls[g % 2])

        ld = load(0)
        zdsc.wait()
        plsc.subcore_barrier()
        adds = None
        for g in range(ngrp):
            nxt = None
            if g + 1 < ngrp:
                if adds is not None:
                    for dsc in adds:
                        dsc.wait()
                    adds = None
                nxt = load(g + 1)
            ld.wait()
            if adds is not None:
                for dsc in adds:
                    dsc.wait()
            buf = (g % 2) * g2
            adds = [pltpu.async_copy(
                msg_v.at[buf + j], acc_sh.at[idx_v.at[g * g2 + j]],
                semas[g % 2], add=True) for j in range(g2)]
            ld = nxt
        for dsc in adds:
            dsc.wait()
        plsc.subcore_barrier()
        pltpu.sync_copy(acc_sh.at[pl.ds(row0, rows_per_sub)],
                        out_hbm.at[pl.ds(c * n + row0, rows_per_sub)])

    return scatter(msg3, dst2, zeros_nh)


def kernel(node_attr, edge_index, edge_attr, node_to_graph, select_reactant,
           num_reactant_batch, num_product_batch,
           W_proj, b_proj, W_bond, b_bond, gnn_bias,
           W_ih, W_hh, b_ih, b_hh, W_sp, b_sp, prelu_a):
    n, d_node = node_attr.shape
    e = edge_index.shape[1]
    d_edge = edge_attr.shape[1]
    h = W_proj.shape[1]
    k = d_edge + 1
    b = num_reactant_batch.shape[0]
    g = select_reactant.shape[0]
    d_hid = W_sp.shape[1]

    nchunk = e // _CH
    src2 = edge_index[0].reshape(nchunk, _CH).astype(jnp.int32)
    dst2 = edge_index[1].reshape(nchunk, _CH).astype(jnp.int32)

    eye4 = jnp.eye(4, dtype=F32)
    wb = W_bond.reshape(d_edge, h, h)
    wk_stack = jax.vmap(lambda m: jnp.kron(eye4, m))(wb)
    ek_stack = jnp.asarray(np.stack([
        np.kron(np.eye(4, dtype=np.float32),
                np.eye(d_edge, dtype=np.float32)[:, [kk]]
                @ np.ones((1, h), np.float32))
        for kk in range(d_edge)]))
    bb = b_bond.reshape(h, h)
    b_big = jax.scipy.linalg.block_diag(bb, bb, bb, bb)

    w_proj4 = jax.scipy.linalg.block_diag(*([W_proj] * 4))
    b_proj4 = jnp.tile(b_proj, (4,)).reshape(1, 4 * h)
    w_ih3 = W_ih.T.reshape(h, 3, h)
    w_hh3 = W_hh.T.reshape(h, 3, h)
    w_ih4 = jnp.concatenate(
        [jax.scipy.linalg.block_diag(*([w_ih3[:, gg, :]] * 4)) for gg in range(3)],
        axis=1)
    w_hh4 = jnp.concatenate(
        [jax.scipy.linalg.block_diag(*([w_hh3[:, gg, :]] * 4)) for gg in range(3)],
        axis=1)
    b_ih4 = jnp.tile(b_ih.reshape(3, 1, h), (1, 4, 1)).reshape(1, 12 * h)
    b_hh4 = jnp.tile(b_hh.reshape(3, 1, h), (1, 4, 1)).reshape(1, 12 * h)
    gbias4 = jnp.tile(gnn_bias, (4,)).reshape(1, 4 * h)

    h0_4 = _proj(node_attr.reshape(n // 4, 4 * d_node), w_proj4, b_proj4)

    zeros_nh = jnp.zeros((n, h), F32)

    hid4 = h0_4
    ea4 = edge_attr.reshape(e // 4, 4 * d_edge)
    for step in range(3):
        hsrc3 = _sc_gather(hid4.reshape(n, h), src2)
        msg4 = _msg(hsrc3.reshape(e // 4, 4 * h), ea4, wk_stack, ek_stack,
                    b_big)
        aggp4 = _sc_scatter(msg4.reshape(nchunk, _CH, h), dst2, zeros_nh)
        hid4 = _gru(aggp4.reshape(2, n // 4, 4 * h), hid4, w_ih4, w_hh4,
                    b_ih4, b_hh4, gbias4)

    r_idx = jnp.nonzero(select_reactant, size=b)[0]
    p_idx = jnp.nonzero(jnp.logical_not(select_reactant), size=b)[0]
    seg_r = jnp.repeat(jnp.arange(b), num_reactant_batch, total_repeat_length=b)
    seg_p = jnp.repeat(jnp.arange(b), num_product_batch, total_repeat_length=b)
    ar = jnp.arange(b)[None, :]
    ag = jnp.arange(g)[None, :]
    m_r = jnp.dot((seg_r[:, None] == ar).astype(F32).T,
                  (r_idx[:, None] == ag).astype(F32))
    m_p = jnp.dot((seg_p[:, None] == ar).astype(F32).T,
                  (p_idx[:, None] == ag).astype(F32))

    ids3 = node_to_graph.astype(jnp.int32).reshape(n // 1000, 1, 1000)
    out = _pool(hid4.reshape(n, h), h0_4.reshape(n, h), ids3,
                W_sp[:h], W_sp[h:], b_sp.reshape(1, d_hid),
                m_r, m_p, jnp.reshape(prelu_a, (1, 1)))
    return out
